# Initial kernel scaffold; baseline (speedup 1.0000x reference)
import jax
import jax.numpy as jnp
from jax.experimental import pallas as pl


def kernel(x, d_indices, d_values, weight, filt, bias):
    raise NotImplementedError("write your pallas kernel here")


# speedup vs baseline: 5.6870x; 5.6870x over previous
import functools

import jax
import jax.numpy as jnp
from jax import lax
from jax.experimental import pallas as pl
from jax.experimental.pallas import tpu as pltpu
from jax.experimental.pallas import tpu_sc as plsc

_N = 10000
_D = 128
_NNZ = 160000
_NLIVE = 3
_NT = 32
_CH = 128
_EPT = 4992
_NCH = _EPT // _CH
_REM_BASE = _EPT * _NT
_NP = 10240
_RPS = _NP // 16
_DCH = 128


def _mm_body(x_ref, w_ref, o_ref):
    o_ref[:, :] = jnp.dot(x_ref[:, :], w_ref[:, :],
                          preferred_element_type=jnp.float32)


def _matmul(x, w):
    return pl.pallas_call(
        _mm_body,
        grid=(10,),
        in_specs=[pl.BlockSpec((_N // 10, _D), lambda i: (i, 0)),
                  pl.BlockSpec((_D, _D), lambda i: (0, 0))],
        out_specs=pl.BlockSpec((_N // 10, _D), lambda i: (i, 0)),
        out_shape=jax.ShapeDtypeStruct((_N, _D), jnp.float32),
    )(x, w)


def _merge_body(a_ref, o_ref):
    o_ref[:, :] = a_ref[0] + a_ref[1]


def _merge(yp):
    rows = _NLIVE * _NP
    blk = 1024
    return pl.pallas_call(
        _merge_body,
        grid=(rows // blk,),
        in_specs=[pl.BlockSpec((2, blk, _D), lambda i: (0, i, 0))],
        out_specs=pl.BlockSpec((blk, _D), lambda i: (i, 0)),
        out_shape=jax.ShapeDtypeStruct((rows, _D), jnp.float32),
    )(yp)


def _final_body(a_ref, b_ref, o_ref):
    o_ref[:, :] = a_ref[0] + a_ref[1] + b_ref[:, :]


def _final(op, bias2d):
    blk = 1000
    return pl.pallas_call(
        _final_body,
        grid=(_N // blk,),
        in_specs=[pl.BlockSpec((2, blk, _D), lambda i: (0, i, 0)),
                  pl.BlockSpec((1, _D), lambda i: (0, 0))],
        out_specs=pl.BlockSpec((blk, _D), lambda i: (i, 0)),
        out_shape=jax.ShapeDtypeStruct((_N, _D), jnp.float32),
    )(op, bias2d)


_MESH = dict(core_axis_name="c", subcore_axis_name="s")


def _zero_buf(buf):
    def zrow(i, carry):
        for j in range(_D // 16):
            buf[i, pl.ds(j * 16, 16)] = jnp.zeros((16,), jnp.float32)
        return carry
    lax.fori_loop(0, _DCH, zrow, 0)


def _scale_rows(gbuf, vals_v):
    def erow(e, carry):
        e16 = jnp.full((16,), 0, jnp.int32) + e
        s16 = plsc.load_gather(vals_v, [e16])
        for c in range(_D // 16):
            gbuf[e, pl.ds(c * 16, 16)] = gbuf[e, pl.ds(c * 16, 16)] * s16
        return carry
    lax.fori_loop(0, _CH, erow, 0)


def _sc_stage1(h, didx, dvals, filt_flat):
    mesh = plsc.VectorSubcoreMesh(**_MESH)

    @functools.partial(
        pl.kernel,
        out_type=jax.ShapeDtypeStruct((2, _NLIVE, _NP, _D), jnp.float32),
        mesh=mesh,
        compiler_params=pltpu.CompilerParams(needs_layout_passes=False),
        scratch_types=[
            pltpu.VMEM_SHARED((_NP, _D), jnp.float32),
            pltpu.VMEM((_CH,), jnp.int32),
            pltpu.VMEM((_CH,), jnp.int32),
            pltpu.VMEM((_CH,), jnp.float32),
            pltpu.VMEM((_CH, _D), jnp.float32),
            pltpu.VMEM((_N,), jnp.float32),
            pltpu.SemaphoreType.DMA,
        ],
    )
    def k(h_hbm, didx_hbm, dvals_hbm, filt_hbm, yp_hbm,
          ysp, rows_v, cols_v, vals_v, gbuf, filt_v, sem):
        cid = lax.axis_index("c")
        sid = lax.axis_index("s")
        tid = cid * 16 + sid

        _zero_buf(gbuf)
        for i in range(_RPS // _DCH):
            pltpu.sync_copy(gbuf, ysp.at[pl.ds(sid * _RPS + i * _DCH, _DCH)])
        plsc.subcore_barrier()

        for mm in (1, 2, 3):
            pltpu.sync_copy(filt_hbm.at[pl.ds(mm * _N, _N)], filt_v)

            def do_chunk(off):
                pltpu.sync_copy(didx_hbm.at[mm, 0, pl.ds(off, _CH)], rows_v)
                pltpu.sync_copy(didx_hbm.at[mm, 1, pl.ds(off, _CH)], cols_v)
                pltpu.sync_copy(dvals_hbm.at[mm, pl.ds(off, _CH)], vals_v)
                pltpu.async_copy(h_hbm.at[cols_v], gbuf, sem).wait()
                for j in range(_CH // 16):
                    r16 = rows_v[pl.ds(j * 16, 16)]
                    f16 = plsc.load_gather(filt_v, [r16])
                    vals_v[pl.ds(j * 16, 16)] = vals_v[pl.ds(j * 16, 16)] * f16
                _scale_rows(gbuf, vals_v)
                pltpu.sync_copy(gbuf, ysp.at[rows_v], add=True)

            def cbody(g, carry):
                do_chunk(tid * _EPT + g * _CH)
                return carry
            lax.fori_loop(0, _NCH, cbody, 0)

            @pl.when(tid < 2)
            def _():
                do_chunk(_REM_BASE + tid * _CH)

            plsc.subcore_barrier()
            _zero_buf(gbuf)
            for i in range(_RPS // _DCH):
                start = sid * _RPS + i * _DCH
                pltpu.sync_copy(ysp.at[pl.ds(start, _DCH)],
                                yp_hbm.at[cid, mm - 1, pl.ds(start, _DCH)])
                pltpu.sync_copy(gbuf, ysp.at[pl.ds(start, _DCH)])
            plsc.subcore_barrier()

    return k(h, didx, dvals, filt_flat)


def _sc_stage2(ym, didx, dvals):
    mesh = plsc.VectorSubcoreMesh(**_MESH)

    @functools.partial(
        pl.kernel,
        out_type=jax.ShapeDtypeStruct((2, _NP, _D), jnp.float32),
        mesh=mesh,
        compiler_params=pltpu.CompilerParams(needs_layout_passes=False),
        scratch_types=[
            pltpu.VMEM_SHARED((_NP, _D), jnp.float32),
            pltpu.VMEM((_CH,), jnp.int32),
            pltpu.VMEM((_CH,), jnp.int32),
            pltpu.VMEM((_CH,), jnp.float32),
            pltpu.VMEM((_CH, _D), jnp.float32),
            pltpu.SemaphoreType.DMA,
        ],
    )
    def k(ym_hbm, didx_hbm, dvals_hbm, op_hbm,
          osp, rows_v, cols_v, vals_v, gbuf, sem):
        cid = lax.axis_index("c")
        sid = lax.axis_index("s")
        tid = cid * 16 + sid

        _zero_buf(gbuf)
        for i in range(_RPS // _DCH):
            pltpu.sync_copy(gbuf, osp.at[pl.ds(sid * _RPS + i * _DCH, _DCH)])
        plsc.subcore_barrier()

        for mm in (1, 2, 3):
            yoff = (mm - 1) * _NP

            def do_chunk(off):
                pltpu.sync_copy(didx_hbm.at[mm, 0, pl.ds(off, _CH)], rows_v)
                pltpu.sync_copy(didx_hbm.at[mm, 1, pl.ds(off, _CH)], cols_v)
                pltpu.sync_copy(dvals_hbm.at[mm, pl.ds(off, _CH)], vals_v)
                for j in range(_CH // 16):
                    c16 = cols_v[pl.ds(j * 16, 16)]
                    cols_v[pl.ds(j * 16, 16)] = c16 + yoff
                pltpu.async_copy(ym_hbm.at[cols_v], gbuf, sem).wait()
                _scale_rows(gbuf, vals_v)
                pltpu.sync_copy(gbuf, osp.at[rows_v], add=True)

            def cbody(g, carry):
                do_chunk(tid * _EPT + g * _CH)
                return carry
            lax.fori_loop(0, _NCH, cbody, 0)

            @pl.when(tid < 2)
            def _():
                do_chunk(_REM_BASE + tid * _CH)

        plsc.subcore_barrier()
        for i in range(_RPS // _DCH):
            start = sid * _RPS + i * _DCH
            pltpu.sync_copy(osp.at[pl.ds(start, _DCH)],
                            op_hbm.at[cid, pl.ds(start, _DCH)])

    return k(ym, didx, dvals)


def kernel(x, d_indices, d_values, weight, filt, bias):
    h = _matmul(x, weight)
    yp = _sc_stage1(h, d_indices, d_values, filt.reshape(-1))
    ym = _merge(yp.reshape(2, _NLIVE * _NP, _D))
    op = _sc_stage2(ym, d_indices, d_values)
    return _final(op, bias.reshape(1, _D))
